# Initial kernel scaffold; baseline (speedup 1.0000x reference)
import jax
import jax.numpy as jnp
from jax.experimental import pallas as pl


def kernel(x):
    raise NotImplementedError("write your pallas kernel here")


# speedup vs baseline: 11.9755x; 11.9755x over previous
import functools

import jax
import jax.numpy as jnp
from jax.experimental import pallas as pl

_TOPK_FRAC = 0.1


def _count_ge(bits, thr):
    return jnp.sum((bits >= thr).astype(jnp.int32), axis=1, keepdims=True)


def _value_search(bits, want):
    def it(i, p):
        cand = p | ((1 << 30) >> i)
        ge = _count_ge(bits, cand)
        return jnp.where(ge >= want, cand, p)

    return jax.lax.fori_loop(0, 31, it, jnp.zeros(want.shape, jnp.int32))


def _index_search(midx, want):
    def it(i, p):
        cand = p | ((1 << 15) >> i)
        c = jnp.sum((midx < cand).astype(jnp.int32), axis=1, keepdims=True)
        return jnp.where(c < want, cand, p)

    return jax.lax.fori_loop(0, 16, it, jnp.zeros(want.shape, jnp.int32))


def _body(x_ref, o_ref, *, k, n_cols):
    xv = x_ref[...]
    bits = jax.lax.bitcast_convert_type(xv, jnp.int32) & jnp.int32(0x7FFFFFFF)
    rows = xv.shape[0]
    kvec = jnp.full((rows, 1), k, jnp.int32)

    vstar = _value_search(bits, kvec)
    g = jnp.sum((bits > vstar).astype(jnp.int32), axis=1, keepdims=True)
    idx = jax.lax.broadcasted_iota(jnp.int32, bits.shape, 1)
    midx_v = jnp.where(bits == vstar, idx, jnp.int32(n_cols))
    t = _index_search(midx_v, kvec - g)
    ustar = _value_search(bits, jnp.int32(n_cols) - t)
    lcnt = jnp.sum((bits < ustar).astype(jnp.int32), axis=1, keepdims=True)
    rp = t - lcnt
    match_u = bits == ustar
    midx_u = jnp.where(match_u, idx, jnp.int32(n_cols))
    j_cut = _index_search(midx_u, rp)

    keep = (bits < ustar) | (match_u & (idx <= j_cut) & (rp >= 1))
    o_ref[...] = xv * keep.astype(jnp.float32)


def kernel(x):
    n, c, h, w = x.shape
    n_cols = h * w
    k = int(max(1, _TOPK_FRAC * h * w))
    rows = n * c
    block_rows = 8
    while rows % block_rows:
        block_rows //= 2
    xr = x.reshape(rows, n_cols)

    out = pl.pallas_call(
        functools.partial(_body, k=k, n_cols=n_cols),
        grid=(rows // block_rows,),
        in_specs=[pl.BlockSpec((block_rows, n_cols), lambda i: (i, 0))],
        out_specs=pl.BlockSpec((block_rows, n_cols), lambda i: (i, 0)),
        out_shape=jax.ShapeDtypeStruct((rows, n_cols), jnp.float32),
    )(xr)
    return out.reshape(n, c, h, w)


# speedup vs baseline: 13.9013x; 1.1608x over previous
import functools

import jax
import jax.numpy as jnp
from jax.experimental import pallas as pl

_TOPK_FRAC = 0.1


def _i16_as_bf16(v):
    return jax.lax.bitcast_convert_type(v.astype(jnp.int16), jnp.bfloat16)


def _count_mask(m, dtype=jnp.bfloat16):
    ones = jnp.ones((m.shape[1], 1), dtype)
    c = jax.lax.dot_general(
        m.astype(dtype),
        ones,
        (((1,), (0,)), ((), ())),
        preferred_element_type=jnp.float32,
    )
    return c.astype(jnp.int32)


def _count_ge(plane, thr, n_part):
    del n_part
    return _count_mask(plane >= thr)


def _count_gt(plane, thr, n_part):
    del n_part
    return _count_mask(plane > thr)


def _staged_value_search(hi_bf, b1, b2, want, n_part):
    shape = want.shape

    def it_a(i, p):
        cand = p | ((1 << 14) >> i)
        ge = _count_ge(hi_bf, _i16_as_bf16(cand), n_part)
        return jnp.where(ge >= want, cand, p)

    p_hi = jax.lax.fori_loop(0, 15, it_a, jnp.zeros(shape, jnp.int32))
    p_hi_bf = _i16_as_bf16(p_hi)
    g1 = _count_gt(hi_bf, p_hi_bf, n_part)

    eq1 = (hi_bf == p_hi_bf).astype(jnp.bfloat16)
    ab1 = eq1 * b1 + eq1 - jnp.bfloat16(1.0)
    want2 = want - g1

    def it_b1(i, p):
        cand = p | ((1 << 7) >> i)
        ge = _count_ge(ab1, cand.astype(jnp.bfloat16), n_part)
        return jnp.where(ge >= want2, cand, p)

    p_b1 = jax.lax.fori_loop(0, 8, it_b1, jnp.zeros(shape, jnp.int32))
    p_b1_bf = p_b1.astype(jnp.bfloat16)
    g2 = _count_gt(ab1, p_b1_bf, n_part)

    eq2 = (ab1 == p_b1_bf).astype(jnp.bfloat16)
    ab2 = eq2 * b2 + eq2 - jnp.bfloat16(1.0)
    want3 = want2 - g2

    def it_b2(i, p):
        cand = p | ((1 << 7) >> i)
        ge = _count_ge(ab2, cand.astype(jnp.bfloat16), n_part)
        return jnp.where(ge >= want3, cand, p)

    p_b2 = jax.lax.fori_loop(0, 8, it_b2, jnp.zeros(shape, jnp.int32))
    p_b2_bf = p_b2.astype(jnp.bfloat16)
    g3 = _count_gt(ab2, p_b2_bf, n_part)
    ge3 = _count_ge(ab2, p_b2_bf, n_part)

    vstar = (p_hi << 16) | (p_b1 << 8) | p_b2
    return vstar, g1 + g2 + g3, g1 + g2 + ge3


def _index_search(midx, want, n_part):
    shape = want.shape

    def it(i, p):
        cand = p | ((1 << 15) >> i)
        c = _count_mask(midx < cand.astype(jnp.float32), jnp.float32)
        return jnp.where(c < want, cand, p)

    return jax.lax.fori_loop(0, 16, it, jnp.zeros(shape, jnp.int32))


def _body(x_ref, o_ref, *, k, n_cols):
    xv = x_ref[...]
    rows = xv.shape[0]
    if n_cols % 256 == 0 and n_cols // 256 <= 256:
        n_part = n_cols // 256
    else:
        n_part = 1
    bits = jax.lax.bitcast_convert_type(xv, jnp.int32) & jnp.int32(0x7FFFFFFF)
    hi_bf = _i16_as_bf16(bits >> 16)
    b1 = ((bits >> 8) & 0xFF).astype(jnp.bfloat16)
    b2 = (bits & 0xFF).astype(jnp.bfloat16)
    kvec = jnp.full((rows, 1), k, jnp.int32)

    vstar, gt_v, _ = _staged_value_search(hi_bf, b1, b2, kvec, n_part)

    r = kvec - gt_v
    idx_i = jax.lax.broadcasted_iota(jnp.int32, (rows, n_cols), 1)
    idx_f = idx_i.astype(jnp.float32)
    midx_v = jnp.where(bits == vstar, idx_f, jnp.float32(n_cols))
    t = jax.lax.cond(
        jnp.any(r > 1),
        lambda: _index_search(midx_v, r, n_part),
        lambda: jnp.min(midx_v, axis=1, keepdims=True).astype(jnp.int32),
    )

    ustar, _, ge_u = _staged_value_search(
        hi_bf, b1, b2, jnp.int32(n_cols) - t, n_part
    )
    lcnt = jnp.int32(n_cols) - ge_u
    rp = t - lcnt
    match_u = bits == ustar
    midx_u = jnp.where(match_u, idx_f, jnp.float32(n_cols))
    j_cut = jax.lax.cond(
        jnp.any(rp > 1),
        lambda: _index_search(midx_u, rp, n_part),
        lambda: jnp.min(midx_u, axis=1, keepdims=True).astype(jnp.int32),
    )

    ustar_f = jax.lax.bitcast_convert_type(ustar, jnp.float32)
    keep = (jnp.abs(xv) < ustar_f) | (match_u & (idx_i <= j_cut) & (rp >= 1))
    o_ref[...] = xv * keep.astype(jnp.float32)


def kernel(x):
    n, c, h, w = x.shape
    n_cols = h * w
    k = int(max(1, _TOPK_FRAC * h * w))
    rows = n * c
    block_rows = 32
    while rows % block_rows:
        block_rows //= 2
    xr = x.reshape(rows, n_cols)

    out = pl.pallas_call(
        functools.partial(_body, k=k, n_cols=n_cols),
        grid=(rows // block_rows,),
        in_specs=[pl.BlockSpec((block_rows, n_cols), lambda i: (i, 0))],
        out_specs=pl.BlockSpec((block_rows, n_cols), lambda i: (i, 0)),
        out_shape=jax.ShapeDtypeStruct((rows, n_cols), jnp.float32),
    )(xr)
    return out.reshape(n, c, h, w)


# speedup vs baseline: 19.9586x; 1.4357x over previous
import functools

import jax
import jax.numpy as jnp
from jax.experimental import pallas as pl

_TOPK_FRAC = 0.1


def _psum(m, part_dtype):
    rows, n = m.shape
    if n % 128 == 0 and n > 128:
        mm = m.astype(part_dtype)
        part = jnp.sum(mm.reshape(rows, n // 128, 128), axis=1)
        return jnp.sum(part.astype(jnp.int32), axis=1, keepdims=True)
    return jnp.sum(m.astype(jnp.int32), axis=1, keepdims=True)


def _value_search(hi, lo, want):
    shape = want.shape

    def it_a(i, p):
        cand_u = p | ((1 << 15) >> i)
        cand = cand_u.astype(jnp.int16) ^ jnp.int16(-0x8000)
        ge = _psum(hi >= cand, jnp.int16)
        return jnp.where(ge >= want, cand_u, p)

    p_hi = jax.lax.fori_loop(0, 16, it_a, jnp.zeros(shape, jnp.int32))
    p_hi16 = p_hi.astype(jnp.int16) ^ jnp.int16(-0x8000)
    g1 = _psum(hi > p_hi16, jnp.int16)

    alo = jnp.where(hi == p_hi16, lo, jnp.int16(-1))
    want2 = want - g1

    def it_b(i, p):
        cand = (p | ((1 << 14) >> i)).astype(jnp.int16)
        ge = _psum(alo >= cand, jnp.int16)
        return jnp.where(ge >= want2, cand.astype(jnp.int32), p)

    p_lo = jax.lax.fori_loop(0, 15, it_b, jnp.zeros(shape, jnp.int32))
    p_lo16 = p_lo.astype(jnp.int16)
    g2 = _psum(alo > p_lo16, jnp.int16)
    ge2 = _psum(alo >= p_lo16, jnp.int16)

    vstar = (p_hi << 15) | p_lo
    return vstar, g1 + g2, g1 + ge2


def _index_search(midx, want):
    shape = want.shape

    def it(i, p):
        cand = p | ((1 << 15) >> i)
        c = _psum(midx < cand.astype(jnp.float32), jnp.float32)
        return jnp.where(c < want, cand, p)

    return jax.lax.fori_loop(0, 16, it, jnp.zeros(shape, jnp.int32))


def _body(x_ref, o_ref, *, k, n_cols):
    xv = x_ref[...]
    rows = xv.shape[0]
    bits = jax.lax.bitcast_convert_type(xv, jnp.int32) & jnp.int32(0x7FFFFFFF)
    hi = (bits >> 15).astype(jnp.int16) ^ jnp.int16(-0x8000)
    lo = (bits & 0x7FFF).astype(jnp.int16)
    kvec = jnp.full((rows, 1), k, jnp.int32)

    vstar, gt_v, _ = _value_search(hi, lo, kvec)

    r = kvec - gt_v
    idx_i = jax.lax.broadcasted_iota(jnp.int32, (rows, n_cols), 1)
    idx_f = idx_i.astype(jnp.float32)
    midx_v = jnp.where(bits == vstar, idx_f, jnp.float32(n_cols))
    t = jax.lax.cond(
        jnp.any(r > 1),
        lambda: _index_search(midx_v, r),
        lambda: jnp.min(midx_v, axis=1, keepdims=True).astype(jnp.int32),
    )

    ustar, _, ge_u = _value_search(hi, lo, jnp.int32(n_cols) - t)
    lcnt = jnp.int32(n_cols) - ge_u
    rp = t - lcnt
    match_u = bits == ustar
    midx_u = jnp.where(match_u, idx_f, jnp.float32(n_cols))
    j_cut = jax.lax.cond(
        jnp.any(rp > 1),
        lambda: _index_search(midx_u, rp),
        lambda: jnp.min(midx_u, axis=1, keepdims=True).astype(jnp.int32),
    )

    ustar_f = jax.lax.bitcast_convert_type(ustar, jnp.float32)
    keep = (jnp.abs(xv) < ustar_f) | (match_u & (idx_i <= j_cut) & (rp >= 1))
    o_ref[...] = xv * keep.astype(jnp.float32)


def kernel(x):
    n, c, h, w = x.shape
    n_cols = h * w
    k = int(max(1, _TOPK_FRAC * h * w))
    rows = n * c
    block_rows = 32
    while rows % block_rows:
        block_rows //= 2
    xr = x.reshape(rows, n_cols)

    out = pl.pallas_call(
        functools.partial(_body, k=k, n_cols=n_cols),
        grid=(rows // block_rows,),
        in_specs=[pl.BlockSpec((block_rows, n_cols), lambda i: (i, 0))],
        out_specs=pl.BlockSpec((block_rows, n_cols), lambda i: (i, 0)),
        out_shape=jax.ShapeDtypeStruct((rows, n_cols), jnp.float32),
    )(xr)
    return out.reshape(n, c, h, w)


# speedup vs baseline: 46.1704x; 2.3133x over previous
import functools

import jax
import jax.numpy as jnp
from jax.experimental import pallas as pl

_TOPK_FRAC = 0.1


def _psum(m, part_dtype):
    rows, n = m.shape
    mm = m.astype(part_dtype)
    if n % 128 == 0 and n > 128:
        acc = None
        while mm.shape[1] > 128:
            cols = mm.shape[1]
            if (cols // 128) % 2:
                tail = mm[:, cols - 128 :]
                acc = tail if acc is None else acc + tail
                mm = mm[:, : cols - 128]
            else:
                half = cols // 2
                mm = mm[:, :half] + mm[:, half:]
        if acc is not None:
            mm = mm + acc
    return jnp.sum(mm.astype(jnp.int32), axis=1, keepdims=True)


def _value_search(hi, lo, want):
    shape = want.shape

    def it_a(i, p):
        cand_u = p | ((1 << 15) >> i)
        cand = cand_u.astype(jnp.int16) ^ jnp.int16(-0x8000)
        ge = _psum(hi >= cand, jnp.int16)
        return jnp.where(ge >= want, cand_u, p)

    p_hi = jax.lax.fori_loop(0, 16, it_a, jnp.zeros(shape, jnp.int32))
    p_hi16 = p_hi.astype(jnp.int16) ^ jnp.int16(-0x8000)
    g1 = _psum(hi > p_hi16, jnp.int16)

    alo = jnp.where(hi == p_hi16, lo, jnp.int16(-1))
    want2 = want - g1

    def it_b(i, p):
        cand = (p | ((1 << 14) >> i)).astype(jnp.int16)
        ge = _psum(alo >= cand, jnp.int16)
        return jnp.where(ge >= want2, cand.astype(jnp.int32), p)

    p_lo = jax.lax.fori_loop(0, 15, it_b, jnp.zeros(shape, jnp.int32))
    p_lo16 = p_lo.astype(jnp.int16)
    g2 = _psum(alo > p_lo16, jnp.int16)
    ge2 = _psum(alo >= p_lo16, jnp.int16)

    vstar = (p_hi << 15) | p_lo
    return vstar, g1 + g2, g1 + ge2


def _index_search(midx, want):
    shape = want.shape

    def it(i, p):
        cand = p | ((1 << 15) >> i)
        c = _psum(midx < cand.astype(jnp.float32), jnp.float32)
        return jnp.where(c < want, cand, p)

    return jax.lax.fori_loop(0, 16, it, jnp.zeros(shape, jnp.int32))


def _body(x_ref, o_ref, *, k, n_cols):
    xv = x_ref[...]
    rows = xv.shape[0]
    bits = jax.lax.bitcast_convert_type(xv, jnp.int32) & jnp.int32(0x7FFFFFFF)
    hi = (bits >> 15).astype(jnp.int16) ^ jnp.int16(-0x8000)
    lo = (bits & 0x7FFF).astype(jnp.int16)
    kvec = jnp.full((rows, 1), k, jnp.int32)

    vstar, gt_v, _ = _value_search(hi, lo, kvec)

    r = kvec - gt_v
    idx_i = jax.lax.broadcasted_iota(jnp.int32, (rows, n_cols), 1)
    idx_f = idx_i.astype(jnp.float32)
    midx_v = jnp.where(bits == vstar, idx_f, jnp.float32(n_cols))
    t = jax.lax.cond(
        jnp.any(r > 1),
        lambda: _index_search(midx_v, r),
        lambda: jnp.min(midx_v, axis=1, keepdims=True).astype(jnp.int32),
    )

    ustar, _, ge_u = _value_search(hi, lo, jnp.int32(n_cols) - t)
    lcnt = jnp.int32(n_cols) - ge_u
    rp = t - lcnt
    match_u = bits == ustar
    midx_u = jnp.where(match_u, idx_f, jnp.float32(n_cols))
    j_cut = jax.lax.cond(
        jnp.any(rp > 1),
        lambda: _index_search(midx_u, rp),
        lambda: jnp.min(midx_u, axis=1, keepdims=True).astype(jnp.int32),
    )

    ustar_f = jax.lax.bitcast_convert_type(ustar, jnp.float32)
    keep = (jnp.abs(xv) < ustar_f) | (match_u & (idx_i <= j_cut) & (rp >= 1))
    o_ref[...] = xv * keep.astype(jnp.float32)


def kernel(x):
    n, c, h, w = x.shape
    n_cols = h * w
    k = int(max(1, _TOPK_FRAC * h * w))
    rows = n * c
    block_rows = 32
    while rows % block_rows:
        block_rows //= 2
    xr = x.reshape(rows, n_cols)

    out = pl.pallas_call(
        functools.partial(_body, k=k, n_cols=n_cols),
        grid=(rows // block_rows,),
        in_specs=[pl.BlockSpec((block_rows, n_cols), lambda i: (i, 0))],
        out_specs=pl.BlockSpec((block_rows, n_cols), lambda i: (i, 0)),
        out_shape=jax.ShapeDtypeStruct((rows, n_cols), jnp.float32),
    )(xr)
    return out.reshape(n, c, h, w)
